# Initial kernel scaffold; baseline (speedup 1.0000x reference)
import jax
import jax.numpy as jnp
from jax.experimental import pallas as pl


def kernel(h, edge_index, edge_type, prev_graph_embeds_forward, time_diff_tensor_forward, prev_graph_embeds_backward, time_diff_tensor_backward, loop_weight, w_comp, bases, time_weight_forward, time_weight_backward, h_bias):
    raise NotImplementedError("write your pallas kernel here")


# speedup vs baseline: 7.5484x; 7.5484x over previous
import jax
import jax.numpy as jnp
from jax import lax
from jax.experimental import pallas as pl
from jax.experimental.pallas import tpu as pltpu
from jax.experimental.pallas import tpu_sc as plsc

N = 10000
E = 320000
D = 128
NUM_RELS = 200
NUM_BASES = 2
INV_TEMP = 0.1

NUM_SC = 2
NUM_TILES = 16
LANES = 16

CHUNK = 80
GROUP = 25
NGROUP = 10
EDGES_PER_TILE = E // NUM_TILES
NP = 10112
ROWS_PER_TILE = NP // NUM_TILES


def _sc_body(h_hbm, src_hbm, dst_hbm, et_hbm, col_hbm, out_hbm,
             src_v, dst_v, et_v, col_v, coef_v, rows_v, scaled_v,
             sem0, sem1, acc_sh):
    cid = lax.axis_index("c")
    sid = lax.axis_index("s")

    pltpu.sync_copy(col_hbm, col_v)
    col_base = cid * NUM_RELS

    def _zfill(i, carry):
        for j in range(D // LANES):
            scaled_v[i, pl.ds(j * LANES, LANES)] = jnp.zeros((LANES,), jnp.float32)
        return carry
    lax.fori_loop(0, CHUNK, _zfill, 0)
    base_row = sid * ROWS_PER_TILE
    for z in range(ROWS_PER_TILE // CHUNK):
        pltpu.sync_copy(scaled_v, acc_sh.at[pl.ds(base_row + z * CHUNK, CHUNK)])
    tail = ROWS_PER_TILE - (ROWS_PER_TILE // CHUNK) * CHUNK
    pltpu.sync_copy(scaled_v.at[pl.ds(0, tail)],
                    acc_sh.at[pl.ds(base_row + ROWS_PER_TILE - tail, tail)])
    plsc.subcore_barrier()

    sems = (sem0, sem1)

    def _gather(j, buf):
        pltpu.async_copy(h_hbm.at[src_v.at[j]], rows_v.at[buf], sems[buf])

    def _process(j, buf):
        pltpu.make_async_copy(h_hbm.at[src_v.at[j]], rows_v.at[buf],
                              sems[buf]).wait()
        for i in range(CHUNK // LANES):
            et16 = et_v[j, pl.ds(i * LANES, LANES)]
            coef_v[pl.ds(i * LANES, LANES)] = plsc.load_gather(
                col_v, [et16 + col_base])
        def _scale(g, carry):
            cf16 = coef_v[pl.ds(g * LANES, LANES)]
            for l in range(LANES):
                c = cf16[l]
                e = g * LANES + l
                for jj in range(D // LANES):
                    sl = pl.ds(jj * LANES, LANES)
                    scaled_v[e, sl] = rows_v[buf, e, sl] * c
            return carry
        lax.fori_loop(0, CHUNK // LANES, _scale, 0)
        @pl.when(j + 2 < GROUP)
        def _():
            _gather(j + 2, buf)
        pltpu.sync_copy(scaled_v, acc_sh.at[dst_v.at[j]], add=True)

    def _group(g, carry):
        pltpu.sync_copy(src_hbm.at[sid, g], src_v)
        pltpu.sync_copy(dst_hbm.at[sid, g], dst_v)
        pltpu.sync_copy(et_hbm.at[sid, g], et_v)
        _gather(0, 0)
        _gather(1, 1)

        def _pair(m, carry2):
            _process(2 * m, 0)
            _process(2 * m + 1, 1)
            return carry2
        lax.fori_loop(0, (GROUP - 1) // 2, _pair, 0)
        _process(GROUP - 1, 0)
        return carry
    lax.fori_loop(0, NGROUP, _group, 0)

    plsc.subcore_barrier()
    pltpu.sync_copy(acc_sh.at[pl.ds(base_row, ROWS_PER_TILE)],
                    out_hbm.at[cid, pl.ds(base_row, ROWS_PER_TILE)])


@jax.jit
def _sc_accumulate(h, src4, dst4, et4, colcat):
    mesh = plsc.VectorSubcoreMesh(core_axis_name="c", subcore_axis_name="s",
                                  num_cores=NUM_SC, num_subcores=NUM_TILES)
    return pl.kernel(
        _sc_body,
        out_type=jax.ShapeDtypeStruct((NUM_BASES, NP, D), jnp.float32),
        mesh=mesh,
        scratch_types=[
            pltpu.VMEM((GROUP, CHUNK), jnp.int32),
            pltpu.VMEM((GROUP, CHUNK), jnp.int32),
            pltpu.VMEM((GROUP, CHUNK), jnp.int32),
            pltpu.VMEM((NUM_RELS * NUM_BASES,), jnp.float32),
            pltpu.VMEM((CHUNK,), jnp.float32),
            pltpu.VMEM((2, CHUNK, D), jnp.float32),
            pltpu.VMEM((CHUNK, D), jnp.float32),
            pltpu.SemaphoreType.DMA,
            pltpu.SemaphoreType.DMA,
            pltpu.VMEM_SHARED((NP, D), jnp.float32),
        ],
        compiler_params=pltpu.CompilerParams(needs_layout_passes=False),
    )(h, src4, dst4, et4, colcat)


def _tc_body(s0_ref, s1_ref, h_ref, pf_ref, tf_ref, pb_ref, tb_ref,
             w_ref, b_ref, o_ref):
    adj_f = pf_ref[...] * jnp.exp(-tf_ref[...] * INV_TEMP)
    adj_b = pb_ref[...] * jnp.exp(-tb_ref[...] * INV_TEMP)
    acc = jnp.dot(s0_ref[...], w_ref[0:D], preferred_element_type=jnp.float32)
    acc = acc + jnp.dot(s1_ref[...], w_ref[D:2 * D],
                        preferred_element_type=jnp.float32)
    acc = acc + jnp.dot(h_ref[...], w_ref[2 * D:3 * D],
                        preferred_element_type=jnp.float32)
    acc = acc + jnp.dot(adj_f, w_ref[3 * D:4 * D],
                        preferred_element_type=jnp.float32)
    acc = acc + jnp.dot(adj_b, w_ref[4 * D:5 * D],
                        preferred_element_type=jnp.float32)
    o_ref[...] = jnp.maximum(acc + b_ref[...], 0.0)


@jax.jit
def _tc_finish(s0, s1, h, pf, tf, pb, tb, w_cat, bias2d):
    R = 2000
    grid = (N // R,)
    row_blk = pl.BlockSpec((R, D), lambda i: (i, 0))
    col1_blk = pl.BlockSpec((R, 1), lambda i: (i, 0))
    full_w = pl.BlockSpec((5 * D, D), lambda i: (0, 0))
    full_b = pl.BlockSpec((1, D), lambda i: (0, 0))
    return pl.pallas_call(
        _tc_body,
        grid=grid,
        in_specs=[row_blk, row_blk, row_blk, row_blk, col1_blk, row_blk,
                  col1_blk, full_w, full_b],
        out_specs=row_blk,
        out_shape=jax.ShapeDtypeStruct((N, D), jnp.float32),
    )(s0, s1, h, pf, tf, pb, tb, w_cat, bias2d)


def kernel(h, edge_index, edge_type, prev_graph_embeds_forward,
           time_diff_tensor_forward, prev_graph_embeds_backward,
           time_diff_tensor_backward, loop_weight, w_comp, bases,
           time_weight_forward, time_weight_backward, h_bias):
    src4 = edge_index[0].reshape(NUM_TILES, NGROUP, GROUP, CHUNK)
    dst4 = edge_index[1].reshape(NUM_TILES, NGROUP, GROUP, CHUNK)
    et4 = edge_type.reshape(NUM_TILES, NGROUP, GROUP, CHUNK)
    colcat = w_comp.T.reshape(-1)
    s = _sc_accumulate(h, src4, dst4, et4, colcat)[:, :N]
    w_cat = jnp.concatenate([bases[0], bases[1], loop_weight,
                             time_weight_forward, time_weight_backward], axis=0)
    return _tc_finish(s[0], s[1], h, prev_graph_embeds_forward,
                      time_diff_tensor_forward, prev_graph_embeds_backward,
                      time_diff_tensor_backward, w_cat,
                      h_bias.reshape(1, D))


# speedup vs baseline: 8.9701x; 1.1884x over previous
import jax
import jax.numpy as jnp
from jax import lax
from jax.experimental import pallas as pl
from jax.experimental.pallas import tpu as pltpu
from jax.experimental.pallas import tpu_sc as plsc

N = 10000
E = 320000
D = 128
DH = 64
NUM_RELS = 200
NUM_BASES = 2
INV_TEMP = 0.1

NUM_SC = 2
NUM_TILES = 16
LANES = 16

CHUNK = 80
GROUP = 25
NGROUP = 10
NP = 10112
ROWS_PER_TILE = NP // NUM_TILES


def _sc_body(h_hbm, src_hbm, dst_hbm, et_hbm, col_hbm, out_hbm,
             src_v, dst_v, et_v, col_v, coef0_v, coef1_v, rows_v,
             scaled0_v, scaled1_v, sem0, sem1, acc0_sh, acc1_sh):
    cid = lax.axis_index("c")
    sid = lax.axis_index("s")

    pltpu.sync_copy(col_hbm, col_v)

    def _zfill(i, carry):
        for j in range(DH // LANES):
            scaled0_v[i, pl.ds(j * LANES, LANES)] = jnp.zeros((LANES,),
                                                              jnp.float32)
        return carry
    lax.fori_loop(0, CHUNK, _zfill, 0)
    base_row = sid * ROWS_PER_TILE
    for acc in (acc0_sh, acc1_sh):
        for z in range(ROWS_PER_TILE // CHUNK):
            pltpu.sync_copy(scaled0_v,
                            acc.at[pl.ds(base_row + z * CHUNK, CHUNK)])
        tail = ROWS_PER_TILE - (ROWS_PER_TILE // CHUNK) * CHUNK
        pltpu.sync_copy(scaled0_v.at[pl.ds(0, tail)],
                        acc.at[pl.ds(base_row + ROWS_PER_TILE - tail, tail)])
    plsc.subcore_barrier()

    sems = (sem0, sem1)

    def _gather(j, buf):
        pltpu.async_copy(h_hbm.at[src_v.at[j]], rows_v.at[buf], sems[buf])

    def _process(j, buf):
        pltpu.make_async_copy(h_hbm.at[src_v.at[j]], rows_v.at[buf],
                              sems[buf]).wait()
        for i in range(CHUNK // LANES):
            et16 = et_v[j, pl.ds(i * LANES, LANES)]
            sl = pl.ds(i * LANES, LANES)
            coef0_v[sl] = plsc.load_gather(col_v, [et16])
            coef1_v[sl] = plsc.load_gather(col_v, [et16 + NUM_RELS])
        def _scale(g, carry):
            cf0 = coef0_v[pl.ds(g * LANES, LANES)]
            cf1 = coef1_v[pl.ds(g * LANES, LANES)]
            for l in range(LANES):
                c0 = cf0[l]
                c1 = cf1[l]
                e = g * LANES + l
                for jj in range(DH // LANES):
                    sl = pl.ds(jj * LANES, LANES)
                    v = rows_v[buf, e, sl]
                    scaled0_v[e, sl] = v * c0
                    scaled1_v[e, sl] = v * c1
            return carry
        lax.fori_loop(0, CHUNK // LANES, _scale, 0)
        @pl.when(j + 2 < GROUP)
        def _():
            _gather(j + 2, buf)
        pltpu.sync_copy(scaled0_v, acc0_sh.at[dst_v.at[j]], add=True)
        pltpu.sync_copy(scaled1_v, acc1_sh.at[dst_v.at[j]], add=True)

    def _group(g, carry):
        pltpu.sync_copy(src_hbm.at[sid, g], src_v)
        pltpu.sync_copy(dst_hbm.at[sid, g], dst_v)
        pltpu.sync_copy(et_hbm.at[sid, g], et_v)

        def _fix(i, carry2):
            for j in range(CHUNK // LANES):
                sl = pl.ds(j * LANES, LANES)
                src_v[i, sl] = src_v[i, sl] + cid
            return carry2
        lax.fori_loop(0, GROUP, _fix, 0)

        _gather(0, 0)
        _gather(1, 1)

        def _pair(m, carry2):
            _process(2 * m, 0)
            _process(2 * m + 1, 1)
            return carry2
        lax.fori_loop(0, (GROUP - 1) // 2, _pair, 0)
        _process(GROUP - 1, 0)
        return carry
    lax.fori_loop(0, NGROUP, _group, 0)

    plsc.subcore_barrier()
    pltpu.sync_copy(acc0_sh.at[pl.ds(base_row, ROWS_PER_TILE)],
                    out_hbm.at[0, cid, pl.ds(base_row, ROWS_PER_TILE)])
    pltpu.sync_copy(acc1_sh.at[pl.ds(base_row, ROWS_PER_TILE)],
                    out_hbm.at[1, cid, pl.ds(base_row, ROWS_PER_TILE)])


@jax.jit
def _sc_accumulate(hI, src4, dst4, et4, colcat):
    mesh = plsc.VectorSubcoreMesh(core_axis_name="c", subcore_axis_name="s",
                                  num_cores=NUM_SC, num_subcores=NUM_TILES)
    return pl.kernel(
        _sc_body,
        out_type=jax.ShapeDtypeStruct((NUM_BASES, NUM_SC, NP, DH),
                                      jnp.float32),
        mesh=mesh,
        scratch_types=[
            pltpu.VMEM((GROUP, CHUNK), jnp.int32),
            pltpu.VMEM((GROUP, CHUNK), jnp.int32),
            pltpu.VMEM((GROUP, CHUNK), jnp.int32),
            pltpu.VMEM((NUM_RELS * NUM_BASES,), jnp.float32),
            pltpu.VMEM((CHUNK,), jnp.float32),
            pltpu.VMEM((CHUNK,), jnp.float32),
            pltpu.VMEM((2, CHUNK, DH), jnp.float32),
            pltpu.VMEM((CHUNK, DH), jnp.float32),
            pltpu.VMEM((CHUNK, DH), jnp.float32),
            pltpu.SemaphoreType.DMA,
            pltpu.SemaphoreType.DMA,
            pltpu.VMEM_SHARED((NP, DH), jnp.float32),
            pltpu.VMEM_SHARED((NP, DH), jnp.float32),
        ],
        compiler_params=pltpu.CompilerParams(needs_layout_passes=False,
                                             use_tc_tiling_on_sc=False),
    )(hI, src4, dst4, et4, colcat)


def _tc_body(s00_ref, s01_ref, s10_ref, s11_ref, h_ref, pf_ref, tf_ref,
             pb_ref, tb_ref, w_ref, b_ref, o_ref):
    adj_f = pf_ref[...] * jnp.exp(-tf_ref[...] * INV_TEMP)
    adj_b = pb_ref[...] * jnp.exp(-tb_ref[...] * INV_TEMP)
    f32 = jnp.float32
    acc = jnp.dot(s00_ref[...], w_ref[0:DH], preferred_element_type=f32)
    acc = acc + jnp.dot(s01_ref[...], w_ref[DH:D], preferred_element_type=f32)
    acc = acc + jnp.dot(s10_ref[...], w_ref[D:D + DH],
                        preferred_element_type=f32)
    acc = acc + jnp.dot(s11_ref[...], w_ref[D + DH:2 * D],
                        preferred_element_type=f32)
    acc = acc + jnp.dot(h_ref[...], w_ref[2 * D:3 * D],
                        preferred_element_type=f32)
    acc = acc + jnp.dot(adj_f, w_ref[3 * D:4 * D], preferred_element_type=f32)
    acc = acc + jnp.dot(adj_b, w_ref[4 * D:5 * D], preferred_element_type=f32)
    o_ref[...] = jnp.maximum(acc + b_ref[...], 0.0)


@jax.jit
def _tc_finish(s00, s01, s10, s11, h, pf, tf, pb, tb, w_cat, bias2d):
    R = 2000
    grid = (N // R,)
    row_blk = pl.BlockSpec((R, D), lambda i: (i, 0))
    half_blk = pl.BlockSpec((R, DH), lambda i: (i, 0))
    col1_blk = pl.BlockSpec((R, 1), lambda i: (i, 0))
    full_w = pl.BlockSpec((5 * D, D), lambda i: (0, 0))
    full_b = pl.BlockSpec((1, D), lambda i: (0, 0))
    return pl.pallas_call(
        _tc_body,
        grid=grid,
        in_specs=[half_blk, half_blk, half_blk, half_blk, row_blk, row_blk,
                  col1_blk, row_blk, col1_blk, full_w, full_b],
        out_specs=row_blk,
        out_shape=jax.ShapeDtypeStruct((N, D), jnp.float32),
    )(s00, s01, s10, s11, h, pf, tf, pb, tb, w_cat, bias2d)


def kernel(h, edge_index, edge_type, prev_graph_embeds_forward,
           time_diff_tensor_forward, prev_graph_embeds_backward,
           time_diff_tensor_backward, loop_weight, w_comp, bases,
           time_weight_forward, time_weight_backward, h_bias):
    hI = h.reshape(2 * N, DH)
    src4 = (edge_index[0] * 2).reshape(NUM_TILES, NGROUP, GROUP, CHUNK)
    dst4 = edge_index[1].reshape(NUM_TILES, NGROUP, GROUP, CHUNK)
    et4 = edge_type.reshape(NUM_TILES, NGROUP, GROUP, CHUNK)
    colcat = w_comp.T.reshape(-1)
    s = _sc_accumulate(hI, src4, dst4, et4, colcat)[:, :, :N]
    w_cat = jnp.concatenate([bases[0], bases[1], loop_weight,
                             time_weight_forward, time_weight_backward],
                            axis=0)
    return _tc_finish(s[0, 0], s[0, 1], s[1, 0], s[1, 1], h,
                      prev_graph_embeds_forward, time_diff_tensor_forward,
                      prev_graph_embeds_backward, time_diff_tensor_backward,
                      w_cat, h_bias.reshape(1, D))
